# baseline (speedup 1.0000x reference)
import functools

import jax
import jax.numpy as jnp
from jax import lax
from jax.experimental import pallas as pl
from jax.experimental.pallas import tpu as pltpu
from jax.experimental.pallas import tpu_sc as plsc

NUM_CORES = 2
NUM_SUBCORES = 16
LANES = 16
NUM_WORKERS = NUM_CORES * NUM_SUBCORES

SUBLANE = 8
LANE128 = 128

T1C = 15
NBUF = 2
UNROLL = 16


@functools.partial(jax.jit, static_argnums=(2, 3))
def _onehot_sc(idx, zsrc, batch, nb_digits):
    rows_per_worker = batch // NUM_WORKERS
    t0n = batch // LANE128
    t0_per_worker = t0n // NUM_WORKERS
    num_t1 = nb_digits // SUBLANE
    starts = list(range(0, num_t1, T1C))
    sizes = [min(T1C, num_t1 - s) for s in starts]
    chunks = len(starts)
    groups = rows_per_worker // LANES

    mesh = plsc.VectorSubcoreMesh(core_axis_name="c", subcore_axis_name="s")

    def body(idx_hbm, zsrc_hbm, out_hbm, idx_v, *bufs_sems):
        bufs, sems = bufs_sems[:NBUF], bufs_sems[NBUF:]
        wid = lax.axis_index("c") * NUM_SUBCORES + lax.axis_index("s")
        row_base = wid * rows_per_worker
        t0_base = wid * t0_per_worker

        pltpu.sync_copy(idx_hbm.at[pl.ds(row_base, rows_per_worker)], idx_v)
        init = [pltpu.async_copy(zsrc_hbm, bufs[b], sems[b])
                for b in range(NBUF)]

        iota = lax.iota(jnp.int32, LANES)
        ones = jnp.full((LANES,), 1.0, jnp.float32)
        zeros = jnp.zeros((LANES,), jnp.float32)

        def span(k):
            lo = starts[k] * SUBLANE
            return lo, lo + sizes[k] * SUBLANE

        def scan_chunk(ko, buf, kz=None):
            olo, ohi = span(ko)
            if kz is not None:
                zlo, zhi = span(kz)

            def it(g0, _):
                for u in range(UNROLL):
                    g = g0 * UNROLL + u
                    col_v = idx_v[pl.ds(g * LANES, LANES)]
                    t1_all = lax.shift_right_logical(col_v, 3)
                    a_v = lax.bitwise_and(col_v, 7)
                    t0_v = jnp.full((LANES,), lax.shift_right_logical(g, 3),
                                    jnp.int32)
                    b_v = lax.shift_left(lax.bitwise_and(g, 7), 4) + iota
                    if kz is not None:
                        mz = jnp.logical_and(col_v >= zlo, col_v < zhi)
                        plsc.store_scatter(
                            buf, [t1_all - starts[kz], t0_v, a_v, b_v],
                            zeros, mask=mz)
                    mo = jnp.logical_and(col_v >= olo, col_v < ohi)
                    plsc.store_scatter(
                        buf, [t1_all - starts[ko], t0_v, a_v, b_v],
                        ones, mask=mo)
                return 0

            lax.fori_loop(0, groups // UNROLL, it, 0, unroll=False)

        descs = [None] * chunks
        for c in range(chunks):
            b = c % NBUF
            if c < NBUF:
                init[b].wait()
                scan_chunk(c, bufs[b])
            else:
                descs[c - NBUF].wait()
                scan_chunk(c, bufs[b], kz=c - NBUF)
            sz = sizes[c]
            src_ref = bufs[b] if sz == T1C else bufs[b].at[pl.ds(0, sz)]
            descs[c] = pltpu.async_copy(
                src_ref,
                out_hbm.at[pl.ds(starts[c], sz),
                           pl.ds(t0_base, t0_per_worker)],
                sems[b])
        for c in range(chunks - NBUF, chunks):
            descs[c].wait()

    f = pl.kernel(
        body,
        out_type=jax.ShapeDtypeStruct((num_t1, t0n, SUBLANE, LANE128),
                                      jnp.float32),
        mesh=mesh,
        scratch_types=(
            [pltpu.VMEM((rows_per_worker,), jnp.int32)]
            + [pltpu.VMEM((T1C, t0_per_worker, SUBLANE, LANE128), jnp.float32)
               for _ in range(NBUF)]
            + [pltpu.SemaphoreType.DMA for _ in range(NBUF)]
        ),
        compiler_params=pltpu.CompilerParams(
            needs_layout_passes=False,
            use_tc_tiling_on_sc=False,
        ),
    )
    t = f(idx, zsrc)
    return t.transpose(1, 3, 0, 2).reshape(batch, nb_digits)


def kernel(onehot_buf, long_tensor, nb_digits):
    del nb_digits
    batch, digits = onehot_buf.shape
    idx = long_tensor.reshape(-1).astype(jnp.int32)
    zsrc = jnp.zeros((T1C, (batch // LANE128) // NUM_WORKERS, SUBLANE,
                      LANE128), jnp.float32)
    return _onehot_sc(idx, zsrc, batch, digits)


# speedup vs baseline: 1.0428x; 1.0428x over previous
import functools

import jax
import jax.numpy as jnp
from jax import lax
from jax.experimental import pallas as pl
from jax.experimental.pallas import tpu as pltpu
from jax.experimental.pallas import tpu_sc as plsc

NUM_CORES = 2
NUM_SUBCORES = 16
LANES = 16
NUM_WORKERS = NUM_CORES * NUM_SUBCORES

SUBLANE = 8
LANE128 = 128

T1C = 15
NBUF = 2
UNROLL = 8


@functools.partial(jax.jit, static_argnums=(2, 3))
def _onehot_sc(idx, zsrc, batch, nb_digits):
    rows_per_worker = batch // NUM_WORKERS
    t0n = batch // LANE128
    t0_per_worker = t0n // NUM_WORKERS
    num_t1 = nb_digits // SUBLANE
    starts = list(range(0, num_t1, T1C))
    sizes = [min(T1C, num_t1 - s) for s in starts]
    chunks = len(starts)
    groups = rows_per_worker // LANES

    mesh = plsc.VectorSubcoreMesh(core_axis_name="c", subcore_axis_name="s")

    def body(idx_hbm, zsrc_hbm, out_hbm, idx_v, *bufs_sems):
        bufs, sems = bufs_sems[:NBUF], bufs_sems[NBUF:]
        wid = lax.axis_index("c") * NUM_SUBCORES + lax.axis_index("s")
        row_base = wid * rows_per_worker
        t0_base = wid * t0_per_worker

        pltpu.sync_copy(idx_hbm.at[pl.ds(row_base, rows_per_worker)], idx_v)
        init = [pltpu.async_copy(zsrc_hbm, bufs[b], sems[b])
                for b in range(NBUF)]

        iota = lax.iota(jnp.int32, LANES)
        ones = jnp.full((LANES,), 1.0, jnp.float32)
        zeros = jnp.zeros((LANES,), jnp.float32)

        def span(k):
            lo = starts[k] * SUBLANE
            return lo, lo + sizes[k] * SUBLANE

        def scan_chunk(ko, buf, kz=None):
            olo, ohi = span(ko)
            if kz is not None:
                zlo, zhi = span(kz)

            def it(g0, _):
                for u in range(UNROLL):
                    g = g0 * UNROLL + u
                    col_v = idx_v[pl.ds(g * LANES, LANES)]
                    t1_all = lax.shift_right_logical(col_v, 3)
                    a_v = lax.bitwise_and(col_v, 7)
                    t0_v = jnp.full((LANES,), lax.shift_right_logical(g, 3),
                                    jnp.int32)
                    b_v = lax.shift_left(lax.bitwise_and(g, 7), 4) + iota
                    if kz is not None:
                        mz = jnp.logical_and(col_v >= zlo, col_v < zhi)
                        plsc.store_scatter(
                            buf, [t1_all - starts[kz], t0_v, a_v, b_v],
                            zeros, mask=mz)
                    mo = jnp.logical_and(col_v >= olo, col_v < ohi)
                    plsc.store_scatter(
                        buf, [t1_all - starts[ko], t0_v, a_v, b_v],
                        ones, mask=mo)
                return 0

            lax.fori_loop(0, groups // UNROLL, it, 0, unroll=False)

        descs = [None] * chunks
        for c in range(chunks):
            b = c % NBUF
            if c < NBUF:
                init[b].wait()
                scan_chunk(c, bufs[b])
            else:
                descs[c - NBUF].wait()
                scan_chunk(c, bufs[b], kz=c - NBUF)
            sz = sizes[c]
            src_ref = bufs[b] if sz == T1C else bufs[b].at[pl.ds(0, sz)]
            descs[c] = pltpu.async_copy(
                src_ref,
                out_hbm.at[pl.ds(starts[c], sz),
                           pl.ds(t0_base, t0_per_worker)],
                sems[b])
        for c in range(chunks - NBUF, chunks):
            descs[c].wait()

    f = pl.kernel(
        body,
        out_type=jax.ShapeDtypeStruct((num_t1, t0n, SUBLANE, LANE128),
                                      jnp.float32),
        mesh=mesh,
        scratch_types=(
            [pltpu.VMEM((rows_per_worker,), jnp.int32)]
            + [pltpu.VMEM((T1C, t0_per_worker, SUBLANE, LANE128), jnp.float32)
               for _ in range(NBUF)]
            + [pltpu.SemaphoreType.DMA for _ in range(NBUF)]
        ),
        compiler_params=pltpu.CompilerParams(
            needs_layout_passes=False,
            use_tc_tiling_on_sc=False,
        ),
    )
    t = f(idx, zsrc)
    return t.transpose(1, 3, 0, 2).reshape(batch, nb_digits)


def kernel(onehot_buf, long_tensor, nb_digits):
    del nb_digits
    batch, digits = onehot_buf.shape
    idx = long_tensor.reshape(-1).astype(jnp.int32)
    zsrc = jnp.zeros((T1C, (batch // LANE128) // NUM_WORKERS, SUBLANE,
                      LANE128), jnp.float32)
    return _onehot_sc(idx, zsrc, batch, digits)
